# baseline (speedup 1.0000x reference)
import functools

import jax
import jax.numpy as jnp
from jax import lax
from jax.experimental import pallas as pl
from jax.experimental.pallas import tpu as pltpu
from jax.experimental.pallas import tpu_sc as plsc

BATCH = 1024
SEQ_LEN = 200
EMB_DIM = 32
OUT_ROWS = SEQ_LEN + 1
HALF = SEQ_LEN // 2
GROUP = 8

_info = plsc.get_sparse_core_info()
_NC, _NS = _info.num_cores, _info.num_subcores
NW = _NC * _NS
BPW = BATCH // NW
NGROUP = BPW // GROUP
BLOCK_ROWS = GROUP * OUT_ROWS


@functools.partial(
    pl.kernel,
    mesh=plsc.VectorSubcoreMesh(core_axis_name="c", subcore_axis_name="s"),
    out_type=jax.ShapeDtypeStruct((BATCH * OUT_ROWS, EMB_DIM), jnp.float32),
    scratch_types=[
        pltpu.VMEM((2 * BPW, HALF), jnp.int32),
        pltpu.VMEM((BLOCK_ROWS, EMB_DIM), jnp.float32),
        pltpu.SemaphoreType.DMA,
    ],
    compiler_params=pltpu.CompilerParams(use_tc_tiling_on_sc=False),
)
def _emb_kernel(idx_hbm, table_hbm, empty_hbm, out_hbm, idx_v, rows_v, sem):
    wid = lax.axis_index("s") * _NC + lax.axis_index("c")
    base = wid * BPW
    pltpu.sync_copy(idx_hbm.at[pl.ds(2 * base, 2 * BPW)], idx_v)
    for j in range(GROUP):
        pltpu.sync_copy(empty_hbm, rows_v.at[pl.ds(j * OUT_ROWS, 1)])

    def body(g, _):
        copies = []
        for j in range(GROUP):
            for h in range(2):
                copies.append(pltpu.async_copy(
                    table_hbm.at[idx_v.at[2 * (g * GROUP + j) + h]],
                    rows_v.at[pl.ds(j * OUT_ROWS + 1 + h * HALF, HALF)],
                    sem))
        for c in copies:
            c.wait()
        pltpu.sync_copy(
            rows_v, out_hbm.at[pl.ds((base + g * GROUP) * OUT_ROWS, BLOCK_ROWS)])
        return 0

    lax.fori_loop(0, NGROUP, body, 0)


def kernel(sentence, table, empty_emb):
    idx = sentence[:, ::-1].astype(jnp.int32).reshape(2 * BATCH, HALF)
    flat = _emb_kernel(idx, table, empty_emb)
    return flat.reshape(BATCH, OUT_ROWS, EMB_DIM)
